# Initial kernel scaffold; baseline (speedup 1.0000x reference)
import jax
import jax.numpy as jnp
from jax.experimental import pallas as pl


def kernel(x, years0, edge_index0, edge_index1, feat_dst, years2, pos_edges, neg_edges, time_w, time_b, Ws0, Wn0, b0, Ws1, Wn1, b1, Wp1, bp1, Wp2, bp2):
    raise NotImplementedError("write your pallas kernel here")


# speedup vs baseline: 2.0849x; 2.0849x over previous
import functools

import jax
import jax.numpy as jnp
from jax import lax
from jax.experimental import pallas as pl
from jax.experimental.pallas import tpu as pltpu
from jax.experimental.pallas import tpu_sc as plsc

N0, N1, N2 = 100000, 50000, 10000
E0, E1, EP = 300000, 100000, 30000
HID = 128

N1P = 50176
N2P = 10112
TRASH = N2
E0P = 307200
E1P = 102400
ESP = 61440

NW = 32
CHUNK = 128
ROWS_PER_SUB = N2P // 16
DEGW = 8

_f32 = jnp.float32


def _clamp_loop(dst_v):
    def cl_body(i, _):
        d = dst_v[pl.ds(i * 16, 16)]
        dst_v[pl.ds(i * 16, 16)] = jnp.minimum(d, TRASH)
        return 0
    lax.fori_loop(0, CHUNK // 16, cl_body, 0)


def _zero_shared_rows(z_v, sh, sid):
    def body(i, _):
        off = pl.multiple_of(sid * ROWS_PER_SUB + i * 64, 8)
        pltpu.sync_copy(z_v, sh.at[pl.ds(off, 64)])
        return 0
    lax.fori_loop(0, ROWS_PER_SUB // 64, body, 0)
    tail = ROWS_PER_SUB % 64
    if tail:
        off = pl.multiple_of(sid * ROWS_PER_SUB + (ROWS_PER_SUB - tail), 8)
        pltpu.sync_copy(z_v.at[pl.ds(0, tail)], sh.at[pl.ds(off, tail)])


def _export_shared_rows(sh, out, cid, sid):
    rb = pl.multiple_of(cid * N2P + sid * ROWS_PER_SUB, 8)
    sb = pl.multiple_of(sid * ROWS_PER_SUB, 8)
    pltpu.sync_copy(sh.at[pl.ds(sb, ROWS_PER_SUB)],
                    out.at[pl.ds(rb, ROWS_PER_SUB)])


def _make_deg():
    mesh = plsc.VectorSubcoreMesh(core_axis_name="c", subcore_axis_name="s")

    @functools.partial(
        pl.kernel,
        out_type=[
            jax.ShapeDtypeStruct((2 * N2P, DEGW), _f32),
            jax.ShapeDtypeStruct((2 * N2P, DEGW), _f32),
        ],
        mesh=mesh,
        scratch_types=[
            pltpu.VMEM((CHUNK,), jnp.int32),
            pltpu.VMEM((CHUNK, DEGW), _f32),
            pltpu.VMEM((64, DEGW), _f32),
            pltpu.VMEM_SHARED((N2P, DEGW), _f32),
            pltpu.VMEM_SHARED((N2P, DEGW), _f32),
        ],
    )
    def degk(dst0, dst1, konst, deg0_out, deg1_out,
             dst_v, ones_v, zd_v, sh0, sh1):
        cid = lax.axis_index("c")
        sid = lax.axis_index("s")
        wid = sid * 2 + cid

        pltpu.sync_copy(konst.at[pl.ds(0, CHUNK)], ones_v)
        pltpu.sync_copy(konst.at[pl.ds(CHUNK, 64)], zd_v)
        _zero_shared_rows(zd_v, sh0, sid)
        _zero_shared_rows(zd_v, sh1, sid)
        plsc.subcore_barrier()

        for dst, sh, ep in ((dst0, sh0, E0P), (dst1, sh1, E1P)):
            per_w = ep // NW
            base0 = wid * per_w

            def chunk_body(ci, _):
                base = pl.multiple_of(base0 + ci * CHUNK, 8)
                pltpu.sync_copy(dst.at[pl.ds(base, CHUNK)], dst_v)
                _clamp_loop(dst_v)
                pltpu.sync_copy(ones_v, sh.at[dst_v], add=True)
                return 0
            lax.fori_loop(0, per_w // CHUNK, chunk_body, 0)

        plsc.subcore_barrier()
        _export_shared_rows(sh0, deg0_out, cid, sid)
        _export_shared_rows(sh1, deg1_out, cid, sid)

    return degk


def _make_seg(ep):
    per_w = ep // NW
    n_chunks = per_w // CHUNK
    mesh = plsc.VectorSubcoreMesh(core_axis_name="c", subcore_axis_name="s")

    @functools.partial(
        pl.kernel,
        out_type=jax.ShapeDtypeStruct((2 * N2P, HID), _f32),
        mesh=mesh,
        scratch_types=[
            pltpu.VMEM((CHUNK,), jnp.int32),
            pltpu.VMEM((CHUNK,), jnp.int32),
            pltpu.VMEM((CHUNK, HID), _f32),
            pltpu.VMEM((64, HID), _f32),
            pltpu.VMEM_SHARED((N2P, HID), _f32),
            pltpu.SemaphoreType.DMA,
        ],
    )
    def seg(table, src, dst, acc_out, src_v, dst_v, rows_v, zb_v, acc_sh, sem):
        cid = lax.axis_index("c")
        sid = lax.axis_index("s")
        wid = sid * 2 + cid
        zero16 = jnp.zeros((16,), _f32)

        def zb_body(r, _):
            for g in range(HID // 16):
                zb_v[r, pl.ds(g * 16, 16)] = zero16
            return 0
        lax.fori_loop(0, 64, zb_body, 0)
        _zero_shared_rows(zb_v, acc_sh, sid)
        plsc.subcore_barrier()

        base0 = wid * per_w

        def chunk_body(ci, _):
            base = pl.multiple_of(base0 + ci * CHUNK, 8)
            pltpu.sync_copy(src.at[pl.ds(base, CHUNK)], src_v)
            pltpu.sync_copy(dst.at[pl.ds(base, CHUNK)], dst_v)
            _clamp_loop(dst_v)
            pltpu.async_copy(table.at[src_v], rows_v, sem).wait()
            pltpu.sync_copy(rows_v, acc_sh.at[dst_v], add=True)
            return 0
        lax.fori_loop(0, n_chunks, chunk_body, 0)

        plsc.subcore_barrier()
        _export_shared_rows(acc_sh, acc_out, cid, sid)

    return seg


def _make_pair_gather(ep):
    per_w = ep // NW
    n_chunks = per_w // CHUNK
    mesh = plsc.VectorSubcoreMesh(core_axis_name="c", subcore_axis_name="s")

    @functools.partial(
        pl.kernel,
        out_type=[
            jax.ShapeDtypeStruct((ep, HID), _f32),
            jax.ShapeDtypeStruct((ep, HID), _f32),
        ],
        mesh=mesh,
        scratch_types=[
            pltpu.VMEM((CHUNK,), jnp.int32),
            pltpu.VMEM((CHUNK, HID), _f32),
            pltpu.SemaphoreType.DMA,
        ],
    )
    def gk(s_tab, d_tab, sidx, didx, es_out, ed_out, idx_v, rows_v, sem):
        cid = lax.axis_index("c")
        sid = lax.axis_index("s")
        wid = sid * 2 + cid
        base0 = wid * per_w

        def chunk_body(ci, _):
            base = pl.multiple_of(base0 + ci * CHUNK, 8)
            pltpu.sync_copy(sidx.at[pl.ds(base, CHUNK)], idx_v)
            pltpu.async_copy(s_tab.at[idx_v], rows_v, sem).wait()
            pltpu.sync_copy(rows_v, es_out.at[pl.ds(base, CHUNK)])
            pltpu.sync_copy(didx.at[pl.ds(base, CHUNK)], idx_v)
            pltpu.async_copy(d_tab.at[idx_v], rows_v, sem).wait()
            pltpu.sync_copy(rows_v, ed_out.at[pl.ds(base, CHUNK)])
            return 0
        lax.fori_loop(0, n_chunks, chunk_body, 0)

    return gk


_BS = 512
_MB = 128


def _dot(a, b):
    return jnp.dot(a, b, preferred_element_type=_f32)


def _prep0_body(x_ref, yf_ref, tw_ref, tb_ref, wna_ref, wnb_ref,
                wsa_ref, wsb_ref, b0_ref, y0_ref, z0_ref):
    te = jnp.cos(yf_ref[...] * tw_ref[...] + tb_ref[...])
    xb = x_ref[...]
    y0_ref[...] = _dot(xb, wna_ref[...]) + _dot(te, wnb_ref[...])
    z0_ref[...] = _dot(xb, wsa_ref[...]) + _dot(te, wsb_ref[...]) + b0_ref[...]


def _mid_body(acca_ref, accb_ref, dega_ref, degb_ref, z0_ref,
              wn1_ref, ws1_ref, b1_ref, y1_ref, z1_ref):
    deg = dega_ref[:, 0] + degb_ref[:, 0]
    recip = 1.0 / jnp.clip(deg, 1.0)
    mean = (acca_ref[...] + accb_ref[...]) * recip[:, None]
    h1 = jnp.maximum(mean + z0_ref[...], 0.0)
    y1_ref[...] = _dot(h1, wn1_ref[...])
    z1_ref[...] = _dot(h1, ws1_ref[...]) + b1_ref[...]


def _post_body(acca_ref, accb_ref, dega_ref, degb_ref, z1_ref, feat_ref,
               yf_ref, tw_ref, tb_ref, wpa_ref, wpb_ref, wpc_ref, bp1_ref,
               s_ref, d_ref):
    deg = dega_ref[:, 0] + degb_ref[:, 0]
    recip = 1.0 / jnp.clip(deg, 1.0)
    h2 = (acca_ref[...] + accb_ref[...]) * recip[:, None] + z1_ref[...]
    te = jnp.cos(yf_ref[...] * tw_ref[...] + tb_ref[...])
    tt = feat_ref[...] + te
    s_ref[...] = _dot(tt, wpa_ref[...])
    d_ref[...] = _dot(h2, wpb_ref[...]) + _dot(tt, wpc_ref[...]) + bp1_ref[...]


def _score_body(es_ref, ed_ref, wp2_ref, bp2_ref, out_ref):
    h = jnp.maximum(es_ref[...] + ed_ref[...], 0.0)
    out_ref[...] = jnp.sum(h * wp2_ref[...], axis=1, keepdims=True) + bp2_ref[...]


def _row_spec(bs, cols):
    return pl.BlockSpec((bs, cols), lambda i: (i, 0))


def _fixed_spec(r, c):
    return pl.BlockSpec((r, c), lambda i: (0, 0))


def kernel(x, years0, edge_index0, edge_index1, feat_dst, years2,
           pos_edges, neg_edges, time_w, time_b,
           Ws0, Wn0, b0, Ws1, Wn1, b1, Wp1, bp1, Wp2, bp2):
    f32 = _f32

    xp = jnp.pad(x[:N1], ((0, N1P - N1), (0, 0)))
    y0f = jnp.pad(years0[:N1].astype(f32), (0, N1P - N1))[:, None]
    tw = time_w[None, :]
    tb = time_b[None, :]

    y0_tab, z0f = pl.pallas_call(
        _prep0_body,
        grid=(N1P // _BS,),
        in_specs=[
            _row_spec(_BS, HID), _row_spec(_BS, 1),
            _fixed_spec(1, HID), _fixed_spec(1, HID),
            _fixed_spec(HID, HID), _fixed_spec(HID, HID),
            _fixed_spec(HID, HID), _fixed_spec(HID, HID),
            _fixed_spec(1, HID),
        ],
        out_specs=[_row_spec(_BS, HID), _row_spec(_BS, HID)],
        out_shape=[
            jax.ShapeDtypeStruct((N1P, HID), f32),
            jax.ShapeDtypeStruct((N1P, HID), f32),
        ],
    )(xp, y0f, tw, tb, Wn0[:HID], Wn0[HID:], Ws0[:HID], Ws0[HID:], b0[None, :])

    konst = jnp.concatenate(
        [jnp.ones((CHUNK, DEGW), f32), jnp.zeros((64, DEGW), f32)])

    src0 = jnp.pad(edge_index0[0], (0, E0P - E0))
    dst0 = jnp.pad(edge_index0[1], (0, E0P - E0), constant_values=TRASH)
    src1 = jnp.pad(edge_index1[0], (0, E1P - E1))
    dst1 = jnp.pad(edge_index1[1], (0, E1P - E1), constant_values=TRASH)

    deg0, deg1 = _make_deg()(dst0, dst1, konst)

    acc0 = _make_seg(E0P)(y0_tab, src0, dst0)

    y1_tab, z1 = pl.pallas_call(
        _mid_body,
        grid=(N2P // _MB,),
        in_specs=[
            _row_spec(_MB, HID), _row_spec(_MB, HID),
            _row_spec(_MB, DEGW), _row_spec(_MB, DEGW),
            _row_spec(_MB, HID),
            _fixed_spec(HID, HID), _fixed_spec(HID, HID), _fixed_spec(1, HID),
        ],
        out_specs=[_row_spec(_MB, HID), _row_spec(_MB, HID)],
        out_shape=[
            jax.ShapeDtypeStruct((N2P, HID), f32),
            jax.ShapeDtypeStruct((N2P, HID), f32),
        ],
    )(acc0[:N2P], acc0[N2P:], deg0[:N2P], deg0[N2P:], z0f[:N2P],
      Wn1, Ws1, b1[None, :])

    acc1 = _make_seg(E1P)(y1_tab, src1, dst1)

    featp = jnp.pad(feat_dst, ((0, N2P - N2), (0, 0)))
    y2f = jnp.pad(years2.astype(f32), (0, N2P - N2))[:, None]

    s_tab, d_tab = pl.pallas_call(
        _post_body,
        grid=(N2P // _MB,),
        in_specs=[
            _row_spec(_MB, HID), _row_spec(_MB, HID),
            _row_spec(_MB, DEGW), _row_spec(_MB, DEGW),
            _row_spec(_MB, HID), _row_spec(_MB, HID), _row_spec(_MB, 1),
            _fixed_spec(1, HID), _fixed_spec(1, HID),
            _fixed_spec(HID, HID), _fixed_spec(HID, HID), _fixed_spec(HID, HID),
            _fixed_spec(1, HID),
        ],
        out_specs=[_row_spec(_MB, HID), _row_spec(_MB, HID)],
        out_shape=[
            jax.ShapeDtypeStruct((N2P, HID), f32),
            jax.ShapeDtypeStruct((N2P, HID), f32),
        ],
    )(acc1[:N2P], acc1[N2P:], deg1[:N2P], deg1[N2P:], z1, featp, y2f,
      tw, tb, Wp1[:HID], Wp1[HID:2 * HID], Wp1[2 * HID:], bp1[None, :])

    sidx = jnp.pad(jnp.concatenate([pos_edges[0], neg_edges[0]]),
                   (0, ESP - 2 * EP))
    didx = jnp.pad(jnp.concatenate([pos_edges[1], neg_edges[1]]),
                   (0, ESP - 2 * EP))
    es, ed = _make_pair_gather(ESP)(s_tab, d_tab, sidx, didx)

    scores = pl.pallas_call(
        _score_body,
        grid=(ESP // _BS,),
        in_specs=[
            _row_spec(_BS, HID), _row_spec(_BS, HID),
            _fixed_spec(1, HID), _fixed_spec(1, 1),
        ],
        out_specs=_row_spec(_BS, 1),
        out_shape=jax.ShapeDtypeStruct((ESP, 1), f32),
    )(es, ed, Wp2[:, 0][None, :], bp2[None, :])

    return scores[:EP, 0], scores[EP:2 * EP, 0]
